# Initial kernel scaffold; baseline (speedup 1.0000x reference)
import functools

import jax
import jax.numpy as jnp
from jax import lax
from jax.experimental import pallas as pl
from jax.experimental.pallas import tpu as pltpu
from jax.experimental.pallas import tpu_sc as plsc

DIM = 128
NC = 2
NS = 16
NW = NC * NS
CH = 128
NBUF = 2


@functools.partial(jax.jit, static_argnames=("b_total",))
def _sc_gather(table, idx2d, b_total):
    per_w = b_total // NW
    nchunk = per_w // CH
    mesh = plsc.VectorSubcoreMesh(core_axis_name="c", subcore_axis_name="s")

    @functools.partial(
        pl.kernel,
        mesh=mesh,
        out_type=jax.ShapeDtypeStruct((b_total, DIM), jnp.float32),
        scratch_types=[
            pltpu.VMEM((nchunk, CH), jnp.int32),
            pltpu.VMEM((NBUF, CH, DIM), jnp.float32),
            pltpu.SemaphoreType.DMA((NBUF,)),
        ],
    )
    def k(table_hbm, idx_hbm, out_hbm, idx_v, rows_v, sems):
        wid = lax.axis_index("s") * NC + lax.axis_index("c")
        row0 = wid * nchunk
        out_base = wid * per_w
        pltpu.sync_copy(idx_hbm.at[pl.ds(row0, nchunk)], idx_v)
        for b in range(NBUF):
            pltpu.make_async_copy(
                table_hbm.at[idx_v.at[b]], rows_v.at[b], sems.at[b]
            ).start()

        def step(s, carry):
            for b in range(NBUF):
                j = s * NBUF + b
                pltpu.make_async_copy(
                    table_hbm.at[idx_v.at[j]], rows_v.at[b], sems.at[b]
                ).wait()
                pltpu.sync_copy(
                    rows_v.at[b], out_hbm.at[pl.ds(out_base + j * CH, CH)]
                )
                nxt = j + NBUF

                @pl.when(nxt < nchunk)
                def _():
                    pltpu.make_async_copy(
                        table_hbm.at[idx_v.at[nxt]], rows_v.at[b], sems.at[b]
                    ).start()

            return carry

        lax.fori_loop(0, nchunk // NBUF, step, 0)

    return k(table, idx2d)


def kernel(word_vector, table):
    b_total = word_vector.size
    idx2d = word_vector.reshape(b_total // CH, CH).astype(jnp.int32)
    out = _sc_gather(table, idx2d, b_total)
    return out.reshape(word_vector.shape + (DIM,))


# speedup vs baseline: 3.3274x; 3.3274x over previous
import functools

import jax
import jax.numpy as jnp
from jax import lax
from jax.experimental import pallas as pl
from jax.experimental.pallas import tpu as pltpu
from jax.experimental.pallas import tpu_sc as plsc

DIM = 128
NC = 2
NS = 16
NW = NC * NS
CH = 128
NBUF = 2


@functools.partial(jax.jit, static_argnames=("b_total",))
def _sc_gather(table, idx, b_total):
    per_w = b_total // NW
    nchunk = per_w // CH
    mesh = plsc.VectorSubcoreMesh(core_axis_name="c", subcore_axis_name="s")

    @functools.partial(
        pl.kernel,
        mesh=mesh,
        out_type=jax.ShapeDtypeStruct((b_total, DIM), jnp.float32),
        scratch_types=[
            pltpu.VMEM((per_w,), jnp.int32),
            pltpu.VMEM((NBUF, CH, DIM), jnp.float32),
            pltpu.SemaphoreType.DMA((NBUF,)),
        ],
    )
    def k(table_hbm, idx_hbm, out_hbm, idx_v, rows_v, sems):
        wid = lax.axis_index("s") * NC + lax.axis_index("c")
        base = wid * per_w
        pltpu.sync_copy(idx_hbm.at[pl.ds(base, per_w)], idx_v)
        for b in range(NBUF):
            pltpu.make_async_copy(
                table_hbm.at[idx_v.at[pl.ds(b * CH, CH)]],
                rows_v.at[b],
                sems.at[b],
            ).start()

        def step(s, carry):
            for b in range(NBUF):
                j = s * NBUF + b
                pltpu.make_async_copy(
                    table_hbm.at[idx_v.at[pl.ds(j * CH, CH)]],
                    rows_v.at[b],
                    sems.at[b],
                ).wait()
                pltpu.sync_copy(
                    rows_v.at[b], out_hbm.at[pl.ds(base + j * CH, CH)]
                )
                nxt = j + NBUF

                @pl.when(nxt < nchunk)
                def _():
                    pltpu.make_async_copy(
                        table_hbm.at[idx_v.at[pl.ds(nxt * CH, CH)]],
                        rows_v.at[b],
                        sems.at[b],
                    ).start()

            return carry

        lax.fori_loop(0, nchunk // NBUF, step, 0)

    return k(table, idx)


def kernel(word_vector, table):
    b_total = word_vector.size
    idx = word_vector.reshape(b_total).astype(jnp.int32)
    out = _sc_gather(table, idx, b_total)
    return out.reshape(word_vector.shape + (DIM,))


# speedup vs baseline: 3.3410x; 1.0041x over previous
import functools

import jax
import jax.numpy as jnp
from jax import lax
from jax.experimental import pallas as pl
from jax.experimental.pallas import tpu as pltpu
from jax.experimental.pallas import tpu_sc as plsc

DIM = 128
NC = 2
NS = 16
NW = NC * NS
CH = 128
NBUF = 5


@functools.partial(jax.jit, static_argnames=("b_total",))
def _sc_gather(table, idx, b_total):
    per_w = b_total // NW
    nchunk = per_w // CH
    mesh = plsc.VectorSubcoreMesh(core_axis_name="c", subcore_axis_name="s")

    @functools.partial(
        pl.kernel,
        mesh=mesh,
        out_type=jax.ShapeDtypeStruct((b_total, DIM), jnp.float32),
        scratch_types=[
            pltpu.VMEM((per_w,), jnp.int32),
            pltpu.VMEM((NBUF, CH, DIM), jnp.float32),
            pltpu.SemaphoreType.DMA((NBUF,)),
            pltpu.SemaphoreType.DMA((NBUF,)),
        ],
    )
    def k(table_hbm, idx_hbm, out_hbm, idx_v, rows_v, gsems, osems):
        wid = lax.axis_index("s") * NC + lax.axis_index("c")
        base = wid * per_w
        pltpu.sync_copy(idx_hbm.at[pl.ds(base, per_w)], idx_v)
        for b in range(NBUF):
            pltpu.make_async_copy(
                table_hbm.at[idx_v.at[pl.ds(b * CH, CH)]],
                rows_v.at[b],
                gsems.at[b],
            ).start()

        def step(s, carry):
            for b in range(NBUF):
                j = s * NBUF + b
                pltpu.make_async_copy(
                    table_hbm.at[idx_v.at[pl.ds(j * CH, CH)]],
                    rows_v.at[b],
                    gsems.at[b],
                ).wait()
                pltpu.make_async_copy(
                    rows_v.at[b], out_hbm.at[pl.ds(base + j * CH, CH)],
                    osems.at[b],
                ).start()
                nxt = j + NBUF

                @pl.when(nxt < nchunk)
                def _():
                    pltpu.make_async_copy(
                        rows_v.at[b], out_hbm.at[pl.ds(base + j * CH, CH)],
                        osems.at[b],
                    ).wait()
                    pltpu.make_async_copy(
                        table_hbm.at[idx_v.at[pl.ds(nxt * CH, CH)]],
                        rows_v.at[b],
                        gsems.at[b],
                    ).start()

            return carry

        lax.fori_loop(0, nchunk // NBUF, step, 0)
        for b in range(NBUF):
            pltpu.make_async_copy(
                rows_v.at[b],
                out_hbm.at[pl.ds(base + (nchunk - NBUF + b) * CH, CH)],
                osems.at[b],
            ).wait()

    return k(table, idx)


def kernel(word_vector, table):
    b_total = word_vector.size
    idx = word_vector.reshape(b_total).astype(jnp.int32)
    out = _sc_gather(table, idx, b_total)
    return out.reshape(word_vector.shape + (DIM,))


# speedup vs baseline: 5.9673x; 1.7861x over previous
import functools

import jax
import jax.numpy as jnp
from jax import lax
from jax.experimental import pallas as pl
from jax.experimental.pallas import tpu as pltpu
from jax.experimental.pallas import tpu_sc as plsc

DIM = 128
NC = 2
NS = 16
NW = NC * NS
NBUF = 4


@functools.partial(jax.jit, static_argnames=("n_sent", "sent_len"))
def _sc_gather(table, word_vector, n_sent, sent_len):
    per_w = n_sent // NW
    mesh = plsc.VectorSubcoreMesh(core_axis_name="c", subcore_axis_name="s")

    @functools.partial(
        pl.kernel,
        mesh=mesh,
        out_type=jax.ShapeDtypeStruct((n_sent, sent_len, DIM), jnp.float32),
        scratch_types=[
            pltpu.VMEM((per_w, sent_len), jnp.int32),
            pltpu.VMEM((NBUF, sent_len, DIM), jnp.float32),
            pltpu.SemaphoreType.DMA((NBUF,)),
            pltpu.SemaphoreType.DMA((NBUF,)),
        ],
    )
    def k(table_hbm, idx_hbm, out_hbm, idx_v, rows_v, gsems, osems):
        wid = lax.axis_index("s") * NC + lax.axis_index("c")
        base = wid * per_w
        pltpu.sync_copy(idx_hbm.at[pl.ds(base, per_w)], idx_v)
        for b in range(NBUF):
            pltpu.make_async_copy(
                table_hbm.at[idx_v.at[b]], rows_v.at[b], gsems.at[b]
            ).start()

        def step(s, carry):
            for b in range(NBUF):
                j = s * NBUF + b
                pltpu.make_async_copy(
                    table_hbm.at[idx_v.at[j]], rows_v.at[b], gsems.at[b]
                ).wait()
                pltpu.make_async_copy(
                    rows_v.at[b], out_hbm.at[base + j], osems.at[b]
                ).start()
                nxt = j + NBUF

                @pl.when(nxt < per_w)
                def _():
                    pltpu.make_async_copy(
                        rows_v.at[b], out_hbm.at[base + j], osems.at[b]
                    ).wait()
                    pltpu.make_async_copy(
                        table_hbm.at[idx_v.at[nxt]], rows_v.at[b], gsems.at[b]
                    ).start()

            return carry

        lax.fori_loop(0, per_w // NBUF, step, 0)
        for b in range(NBUF):
            pltpu.make_async_copy(
                rows_v.at[b], out_hbm.at[base + per_w - NBUF + b], osems.at[b]
            ).wait()

    return k(table, word_vector)


def kernel(word_vector, table):
    n_sent, sent_len = word_vector.shape
    return _sc_gather(table, word_vector.astype(jnp.int32), n_sent, sent_len)


# speedup vs baseline: 10.8031x; 1.8104x over previous
import functools

import jax
import jax.numpy as jnp
from jax import lax
from jax.experimental import pallas as pl
from jax.experimental.pallas import tpu as pltpu
from jax.experimental.pallas import tpu_sc as plsc

DIM = 128
NC = 2
NS = 16
NW = NC * NS
NBUF = 5


@functools.partial(jax.jit, static_argnames=("n_pos", "n_sent"))
def _sc_gather(table, idx_t, n_pos, n_sent):
    cols = n_sent // NW
    mesh = plsc.VectorSubcoreMesh(core_axis_name="c", subcore_axis_name="s")

    @functools.partial(
        pl.kernel,
        mesh=mesh,
        out_type=jax.ShapeDtypeStruct((n_pos, n_sent, DIM), jnp.float32),
        scratch_types=[
            pltpu.VMEM((n_pos, cols), jnp.int32),
            pltpu.VMEM((NBUF, cols, DIM), jnp.float32),
            pltpu.SemaphoreType.DMA((NBUF,)),
            pltpu.SemaphoreType.DMA((NBUF,)),
        ],
    )
    def k(table_hbm, idx_hbm, out_hbm, idx_v, rows_v, gsems, osems):
        wid = lax.axis_index("s") * NC + lax.axis_index("c")
        col0 = wid * cols
        pltpu.sync_copy(idx_hbm.at[:, pl.ds(col0, cols)], idx_v)
        for b in range(NBUF):
            pltpu.make_async_copy(
                table_hbm.at[idx_v.at[b]], rows_v.at[b], gsems.at[b]
            ).start()

        def step(s, carry):
            for b in range(NBUF):
                j = s * NBUF + b
                pltpu.make_async_copy(
                    table_hbm.at[idx_v.at[j]], rows_v.at[b], gsems.at[b]
                ).wait()
                pltpu.make_async_copy(
                    rows_v.at[b], out_hbm.at[j, pl.ds(col0, cols)], osems.at[b]
                ).start()
                nxt = j + NBUF

                @pl.when(nxt < n_pos)
                def _():
                    pltpu.make_async_copy(
                        rows_v.at[b], out_hbm.at[j, pl.ds(col0, cols)],
                        osems.at[b],
                    ).wait()
                    pltpu.make_async_copy(
                        table_hbm.at[idx_v.at[nxt]], rows_v.at[b], gsems.at[b]
                    ).start()

            return carry

        lax.fori_loop(0, n_pos // NBUF, step, 0)
        for b in range(NBUF):
            pltpu.make_async_copy(
                rows_v.at[b],
                out_hbm.at[n_pos - NBUF + b, pl.ds(col0, cols)],
                osems.at[b],
            ).wait()

    return k(table, idx_t)


def kernel(word_vector, table):
    n_sent, n_pos = word_vector.shape
    idx_t = word_vector.T.astype(jnp.int32)
    out_t = _sc_gather(table, idx_t, n_pos, n_sent)
    return out_t.transpose(1, 0, 2)
